# baseline (speedup 1.0000x reference)
import jax
import jax.numpy as jnp
from jax import lax
from jax.experimental import pallas as pl
from jax.experimental.pallas import tpu as pltpu
from jax.experimental.pallas import tpu_sc as plsc

VOCAB = 1_000_000
D = 16
F = 26
L = 20
B = 4096
TASK_DIM = 128
PER_ROW = F * L
OUT_D = F * D + TASK_DIM

NC, NS = 2, 16
NW = NC * NS
B_W = B // NW
C = 8
N_CHUNK = B_W // C
IDX_CHUNK = C * PER_ROW

TCH = 2048
N_TCH_FULL = VOCAB // TCH
TAIL = VOCAB - N_TCH_FULL * TCH
OUT_OFF = D * TCH


def _transpose_body(table_t_hbm, out_hbm, buf_v):
    wid = lax.axis_index("s") * NC + lax.axis_index("c")
    iota16 = lax.iota(jnp.int32, D)

    def do_chunk(v0, n):
        for d in range(D):
            pltpu.sync_copy(table_t_hbm.at[d, pl.ds(v0, n)],
                            buf_v.at[pl.ds(d * n, n)])
        plane = iota16 * n

        def row_body(r8, carry):
            for u in range(8):
                r = r8 * 8 + u
                col = plsc.load_gather(buf_v, [plane + r])
                buf_v[pl.ds(OUT_OFF + r * D, D)] = col
            return carry

        lax.fori_loop(0, n // 8, row_body, 0)
        pltpu.sync_copy(buf_v.at[pl.ds(OUT_OFF, n * D)],
                        out_hbm.at[pl.ds(v0 * D, n * D)])

    n_mine = (N_TCH_FULL - wid + NW - 1) // NW

    def chunk_loop(it, carry):
        do_chunk((wid + it * NW) * TCH, TCH)
        return carry

    lax.fori_loop(0, n_mine, chunk_loop, 0)

    @pl.when(wid == NW - 1)
    def _tail():
        do_chunk(N_TCH_FULL * TCH, TAIL)


def _gather_body(indices_hbm, task_ids_hbm, table_hbm, task_table_hbm,
                 out_hbm, idx_v, rows_v, out_v, tids_v, task_rows_v, sem):
    wid = lax.axis_index("s") * NC + lax.axis_index("c")
    woff_rows = wid * B_W
    woff_idx = woff_rows * PER_ROW

    pltpu.sync_copy(task_ids_hbm.at[pl.ds(woff_rows, B_W)], tids_v)
    pltpu.async_copy(task_table_hbm.at[tids_v], task_rows_v, sem).wait()

    def chunk_body(g, carry):
        row_base = woff_rows + g * C
        pltpu.sync_copy(
            indices_hbm.at[pl.ds(woff_idx + g * IDX_CHUNK, IDX_CHUNK)], idx_v)
        pltpu.async_copy(table_hbm.at[idx_v], rows_v, sem).wait()
        for c in range(C):
            def field_body(f, carry2):
                base = c * PER_ROW + f * L
                acc = rows_v[base]
                for l in range(1, L):
                    acc = acc + rows_v[base + l]
                out_v[pl.ds(c * OUT_D + f * D, D)] = acc
                return carry2
            lax.fori_loop(0, F, field_body, 0)
            trow = g * C + c
            for r in range(TASK_DIM // 16):
                out_v[pl.ds(c * OUT_D + F * D + r * 16, 16)] = \
                    task_rows_v[trow, pl.ds(r * 16, 16)]
        pltpu.sync_copy(out_v, out_hbm.at[pl.ds(row_base * OUT_D, C * OUT_D)])
        return carry

    lax.fori_loop(0, N_CHUNK, chunk_body, 0)


def kernel(indices, task_ids, main_table, task_table):
    idx_flat = indices.reshape(-1)
    mesh = plsc.VectorSubcoreMesh(core_axis_name="c", subcore_axis_name="s")
    params = pltpu.CompilerParams(use_tc_tiling_on_sc=False,
                                  needs_layout_passes=False)

    transpose = pl.kernel(
        _transpose_body,
        mesh=mesh,
        compiler_params=params,
        out_type=jax.ShapeDtypeStruct((VOCAB * D,), jnp.float32),
        scratch_types=[pltpu.VMEM((2 * D * TCH,), jnp.float32)],
    )
    table_rm = transpose(main_table.T).reshape(VOCAB, D)

    gather = pl.kernel(
        _gather_body,
        mesh=mesh,
        compiler_params=params,
        out_type=jax.ShapeDtypeStruct((B * OUT_D,), jnp.float32),
        scratch_types=[
            pltpu.VMEM((IDX_CHUNK,), jnp.int32),
            pltpu.VMEM((IDX_CHUNK, D), jnp.float32),
            pltpu.VMEM((C * OUT_D,), jnp.float32),
            pltpu.VMEM((B_W,), jnp.int32),
            pltpu.VMEM((B_W, TASK_DIM), jnp.float32),
            pltpu.SemaphoreType.DMA,
        ],
    )
    return gather(idx_flat, task_ids, table_rm, task_table).reshape(B, OUT_D)


# speedup vs baseline: 3.1928x; 3.1928x over previous
import jax
import jax.numpy as jnp
from jax import lax
from jax.experimental import pallas as pl
from jax.experimental.pallas import tpu as pltpu
from jax.experimental.pallas import tpu_sc as plsc

VOCAB = 1_000_000
D = 16
F = 26
L = 20
B = 4096
TASK_DIM = 128
PER_ROW = F * L
OUT_D = F * D + TASK_DIM

NC, NS = 2, 16
NW = NC * NS
B_W = B // NW
C = 4
N_CHUNK = B_W // C
N_PAIR = N_CHUNK // 2
IDX_CHUNK = C * PER_ROW


def _gather_body(indices_hbm, task_ids_hbm, table_hbm, task_table_hbm,
                 out_hbm, idx0, idx1, rows0, rows1, out_v, tids_v,
                 task_rows_v, sem0, sem1, semt):
    wid = lax.axis_index("s") * NC + lax.axis_index("c")
    woff_rows = wid * B_W
    woff_idx = woff_rows * PER_ROW

    pltpu.sync_copy(task_ids_hbm.at[pl.ds(woff_rows, B_W)], tids_v)
    pltpu.async_copy(task_table_hbm.at[tids_v], task_rows_v, semt).wait()

    def reduce_chunk(g, rows_v):
        for c in range(C):
            def field_body(f, carry):
                base = c * PER_ROW + f * L
                acc = rows_v[base]
                for l in range(1, L):
                    acc = acc + rows_v[base + l]
                out_v[pl.ds(c * OUT_D + f * D, D)] = acc
                return carry
            lax.fori_loop(0, F, field_body, 0)
            trow = g * C + c
            for r in range(TASK_DIM // 16):
                out_v[pl.ds(c * OUT_D + F * D + r * 16, 16)] = \
                    task_rows_v[trow, pl.ds(r * 16, 16)]
        row_base = woff_rows + g * C
        pltpu.sync_copy(out_v, out_hbm.at[pl.ds(row_base * OUT_D, C * OUT_D)])

    pltpu.sync_copy(indices_hbm.at[pl.ds(woff_idx, IDX_CHUNK)], idx0)
    pltpu.async_copy(table_hbm.at[idx0], rows0, sem0)

    def pair_body(k, carry):
        g0 = 2 * k
        pltpu.sync_copy(
            indices_hbm.at[pl.ds(woff_idx + (g0 + 1) * IDX_CHUNK, IDX_CHUNK)],
            idx1)
        pltpu.async_copy(table_hbm.at[idx1], rows1, sem1)
        pltpu.make_async_copy(table_hbm.at[idx0], rows0, sem0).wait()
        reduce_chunk(g0, rows0)

        @pl.when(k < N_PAIR - 1)
        def _():
            pltpu.sync_copy(
                indices_hbm.at[
                    pl.ds(woff_idx + (g0 + 2) * IDX_CHUNK, IDX_CHUNK)], idx0)
            pltpu.async_copy(table_hbm.at[idx0], rows0, sem0)
        pltpu.make_async_copy(table_hbm.at[idx1], rows1, sem1).wait()
        reduce_chunk(g0 + 1, rows1)
        return carry

    lax.fori_loop(0, N_PAIR, pair_body, 0)


def kernel(indices, task_ids, main_table, task_table):
    idx_flat = indices.reshape(-1)
    mesh = plsc.VectorSubcoreMesh(core_axis_name="c", subcore_axis_name="s")
    gather = pl.kernel(
        _gather_body,
        mesh=mesh,
        compiler_params=pltpu.CompilerParams(use_tc_tiling_on_sc=False),
        out_type=jax.ShapeDtypeStruct((B * OUT_D,), jnp.float32),
        scratch_types=[
            pltpu.VMEM((IDX_CHUNK,), jnp.int32),
            pltpu.VMEM((IDX_CHUNK,), jnp.int32),
            pltpu.VMEM((IDX_CHUNK, D), jnp.float32),
            pltpu.VMEM((IDX_CHUNK, D), jnp.float32),
            pltpu.VMEM((C * OUT_D,), jnp.float32),
            pltpu.VMEM((B_W,), jnp.int32),
            pltpu.VMEM((B_W, TASK_DIM), jnp.float32),
            pltpu.SemaphoreType.DMA,
            pltpu.SemaphoreType.DMA,
            pltpu.SemaphoreType.DMA,
        ],
    )
    return gather(idx_flat, task_ids, main_table,
                  task_table).reshape(B, OUT_D)
